# Initial kernel scaffold; baseline (speedup 1.0000x reference)
import functools

import jax
import jax.numpy as jnp
from jax import lax
from jax.experimental import pallas as pl
from jax.experimental.pallas import tpu as pltpu
from jax.experimental.pallas import tpu_sc as plsc

N = 10000
E = 320000
D = 128

NC = 2
NS = 16
NW = NC * NS
EW = E // NW
CHUNK = 80
NCHUNKS = EW // CHUNK
RPS = N // NS
ZROWS = 125
DEGW = 16

_mesh = plsc.VectorSubcoreMesh(core_axis_name="c", subcore_axis_name="s")


def _zero_vmem(buf, nrows, ncols):
    zero = jnp.zeros((16,), jnp.float32)

    @pl.loop(0, nrows)
    def _(r):
        for c in range(ncols // 16):
            buf[r, pl.ds(c * 16, 16)] = zero


@functools.partial(
    pl.kernel,
    out_type=jax.ShapeDtypeStruct((NC, N, DEGW), jnp.float32),
    mesh=_mesh,
    scratch_types=[
        pltpu.VMEM((1, CHUNK), jnp.int32),
        pltpu.VMEM((CHUNK, DEGW), jnp.float32),
        pltpu.VMEM((ZROWS, DEGW), jnp.float32),
    ],
)
def _deg_kernel(dst_hbm, out_hbm, didx, ones, zbuf):
    core = lax.axis_index("c")
    sub = lax.axis_index("s")
    wid = sub * NC + core

    def body(acc_ref):
        _zero_vmem(zbuf, ZROWS, DEGW)
        one = jnp.full((16,), 1.0, jnp.float32)

        @pl.loop(0, CHUNK)
        def _(r):
            ones[r, pl.ds(0, 16)] = one

        for k in range(RPS // ZROWS):
            pltpu.sync_copy(zbuf, acc_ref.at[pl.ds(sub * RPS + k * ZROWS, ZROWS)])
        plsc.subcore_barrier()

        @pl.loop(0, NCHUNKS)
        def _(j):
            base = wid * EW + j * CHUNK
            pltpu.sync_copy(dst_hbm.at[pl.ds(base, CHUNK)], didx.at[0])
            pltpu.sync_copy(ones, acc_ref.at[didx.at[0]], add=True)

        plsc.subcore_barrier()
        for k in range(RPS // ZROWS):
            r0 = sub * RPS + k * ZROWS
            pltpu.sync_copy(acc_ref.at[pl.ds(r0, ZROWS)], zbuf)
            pltpu.sync_copy(zbuf, out_hbm.at[core, pl.ds(r0, ZROWS)])

    pl.run_scoped(body, pltpu.VMEM_SHARED((N, DEGW), jnp.float32))


@functools.partial(
    pl.kernel,
    out_type=jax.ShapeDtypeStruct((NC, N, D), jnp.float32),
    mesh=_mesh,
    scratch_types=[
        pltpu.VMEM((1, CHUNK), jnp.int32),
        pltpu.VMEM((1, CHUNK), jnp.int32),
        pltpu.VMEM((CHUNK, D), jnp.float32),
        pltpu.VMEM((ZROWS, D), jnp.float32),
        pltpu.SemaphoreType.DMA,
    ],
)
def _agg_kernel(src_hbm, dst_hbm, xs_hbm, out_hbm, sidx, didx, rows, zbuf, sem):
    core = lax.axis_index("c")
    sub = lax.axis_index("s")
    wid = sub * NC + core

    def body(acc_ref):
        _zero_vmem(zbuf, ZROWS, D)
        for k in range(RPS // ZROWS):
            pltpu.sync_copy(zbuf, acc_ref.at[pl.ds(sub * RPS + k * ZROWS, ZROWS)])
        plsc.subcore_barrier()

        @pl.loop(0, NCHUNKS)
        def _(j):
            base = wid * EW + j * CHUNK
            pltpu.sync_copy(src_hbm.at[pl.ds(base, CHUNK)], sidx.at[0])
            pltpu.sync_copy(dst_hbm.at[pl.ds(base, CHUNK)], didx.at[0])
            pltpu.async_copy(xs_hbm.at[sidx.at[0]], rows, sem).wait()
            pltpu.sync_copy(rows, acc_ref.at[didx.at[0]], add=True)

        plsc.subcore_barrier()
        for k in range(RPS // ZROWS):
            r0 = sub * RPS + k * ZROWS
            pltpu.sync_copy(acc_ref.at[pl.ds(r0, ZROWS)], zbuf)
            pltpu.sync_copy(zbuf, out_hbm.at[core, pl.ds(r0, ZROWS)])

    pl.run_scoped(body, pltpu.VMEM_SHARED((N, D), jnp.float32))


BLK = 1000


def _dinv_of(degp_ref):
    deg = degp_ref[0, :, 0] + degp_ref[1, :, 0] + 1.0
    return lax.rsqrt(deg)


def _tc1_body(x_ref, w_ref, degp_ref, xs_ref):
    dinv = _dinv_of(degp_ref)
    xw = jnp.dot(x_ref[...], w_ref[...], preferred_element_type=jnp.float32)
    xs_ref[...] = xw * dinv[:, None]


def _tc2_body(p_ref, xs1_ref, degp_ref, b_ref, w_ref, xs2_ref):
    dinv = _dinv_of(degp_ref)
    s = p_ref[0] + p_ref[1] + xs1_ref[...]
    h = jnp.maximum(dinv[:, None] * s + b_ref[...], 0.0)
    hw = jnp.dot(h, w_ref[...], preferred_element_type=jnp.float32)
    xs2_ref[...] = hw * dinv[:, None]


def _tc3_body(p_ref, xs2_ref, degp_ref, b_ref, out_ref):
    dinv = _dinv_of(degp_ref)
    s = p_ref[0] + p_ref[1] + xs2_ref[...]
    out_ref[...] = dinv[:, None] * s + b_ref[...]


_row_spec = pl.BlockSpec((BLK, D), lambda i: (i, 0))
_p_spec = pl.BlockSpec((NC, BLK, D), lambda i: (0, i, 0))
_degp_spec = pl.BlockSpec((NC, BLK, DEGW), lambda i: (0, i, 0))
_w_spec = pl.BlockSpec((D, D), lambda i: (0, 0))
_b_spec = pl.BlockSpec((1, D), lambda i: (0, 0))

_tc1 = pl.pallas_call(
    _tc1_body,
    grid=(N // BLK,),
    in_specs=[_row_spec, _w_spec, _degp_spec],
    out_specs=_row_spec,
    out_shape=jax.ShapeDtypeStruct((N, D), jnp.float32),
)

_tc2 = pl.pallas_call(
    _tc2_body,
    grid=(N // BLK,),
    in_specs=[_p_spec, _row_spec, _degp_spec, _b_spec, _w_spec],
    out_specs=_row_spec,
    out_shape=jax.ShapeDtypeStruct((N, D), jnp.float32),
)

_tc3 = pl.pallas_call(
    _tc3_body,
    grid=(N // BLK,),
    in_specs=[_p_spec, _row_spec, _degp_spec, _b_spec],
    out_specs=_row_spec,
    out_shape=jax.ShapeDtypeStruct((N, D), jnp.float32),
)


def kernel(x, edge_index, W1, b1, W2, b2):
    src = edge_index[0].astype(jnp.int32)
    dst = edge_index[1].astype(jnp.int32)
    b1r = b1.reshape(1, D)
    b2r = b2.reshape(1, D)

    degp = _deg_kernel(dst)
    xs1 = _tc1(x, W1, degp)
    p1 = _agg_kernel(src, dst, xs1)
    xs2 = _tc2(p1, xs1, degp, b1r, W2)
    p2 = _agg_kernel(src, dst, xs2)
    return _tc3(p2, xs2, degp, b2r)


# speedup vs baseline: 12.7077x; 12.7077x over previous
import functools

import jax
import jax.numpy as jnp
from jax import lax
from jax.experimental import pallas as pl
from jax.experimental.pallas import tpu as pltpu
from jax.experimental.pallas import tpu_sc as plsc

N = 10000
NPAD = 10240
E = 320000
D = 128

NC = 2
NS = 16
NW = NC * NS
EW = E // NW
CHUNK = 80
NCHUNKS = EW // CHUNK
RPS = NPAD // NS
ZROWS = 128
DEGW = 128

_mesh = plsc.VectorSubcoreMesh(core_axis_name="c", subcore_axis_name="s")


def _zero_vmem(buf, nrows, ncols):
    zero = jnp.zeros((16,), jnp.float32)

    @pl.loop(0, nrows)
    def _(r):
        for c in range(ncols // 16):
            buf[r, pl.ds(c * 16, 16)] = zero


@functools.partial(
    pl.kernel,
    out_type=jax.ShapeDtypeStruct((NC, NPAD, DEGW), jnp.float32),
    mesh=_mesh,
    scratch_types=[
        pltpu.VMEM((1, CHUNK), jnp.int32),
        pltpu.VMEM((CHUNK, DEGW), jnp.float32),
        pltpu.VMEM((ZROWS, DEGW), jnp.float32),
        pltpu.VMEM_SHARED((NPAD, DEGW), jnp.float32),
    ],
)
def _deg_kernel(dst_hbm, out_hbm, didx, ones, zbuf, acc_ref):
    core = lax.axis_index("c")
    sub = lax.axis_index("s")
    wid = sub * NC + core

    if True:
        _zero_vmem(zbuf, ZROWS, DEGW)
        one = jnp.full((16,), 1.0, jnp.float32)

        @pl.loop(0, CHUNK)
        def _(r):
            for c in range(DEGW // 16):
                ones[r, pl.ds(c * 16, 16)] = one

        for k in range(RPS // ZROWS):
            pltpu.sync_copy(zbuf, acc_ref.at[pl.ds(sub * RPS + k * ZROWS, ZROWS)])
        plsc.subcore_barrier()

        @pl.loop(0, NCHUNKS)
        def _(j):
            base = wid * EW + j * CHUNK
            pltpu.sync_copy(dst_hbm.at[pl.ds(base, CHUNK)], didx.at[0])
            pltpu.sync_copy(ones, acc_ref.at[didx.at[0]], add=True)

        plsc.subcore_barrier()
        for k in range(RPS // ZROWS):
            r0 = sub * RPS + k * ZROWS
            pltpu.sync_copy(acc_ref.at[pl.ds(r0, ZROWS)], zbuf)
            pltpu.sync_copy(zbuf, out_hbm.at[core, pl.ds(r0, ZROWS)])


@functools.partial(
    pl.kernel,
    out_type=jax.ShapeDtypeStruct((NC, NPAD, D), jnp.float32),
    mesh=_mesh,
    scratch_types=[
        pltpu.VMEM((1, CHUNK), jnp.int32),
        pltpu.VMEM((1, CHUNK), jnp.int32),
        pltpu.VMEM((CHUNK, D), jnp.float32),
        pltpu.VMEM((ZROWS, D), jnp.float32),
        pltpu.VMEM_SHARED((NPAD, D), jnp.float32),
        pltpu.SemaphoreType.DMA,
    ],
)
def _agg_kernel(src_hbm, dst_hbm, xs_hbm, out_hbm, sidx, didx, rows, zbuf, acc_ref, sem):
    core = lax.axis_index("c")
    sub = lax.axis_index("s")
    wid = sub * NC + core

    if True:
        _zero_vmem(zbuf, ZROWS, D)
        for k in range(RPS // ZROWS):
            pltpu.sync_copy(zbuf, acc_ref.at[pl.ds(sub * RPS + k * ZROWS, ZROWS)])
        plsc.subcore_barrier()

        @pl.loop(0, NCHUNKS)
        def _(j):
            base = wid * EW + j * CHUNK
            pltpu.sync_copy(src_hbm.at[pl.ds(base, CHUNK)], sidx.at[0])
            pltpu.sync_copy(dst_hbm.at[pl.ds(base, CHUNK)], didx.at[0])
            pltpu.async_copy(xs_hbm.at[sidx.at[0]], rows, sem).wait()
            pltpu.sync_copy(rows, acc_ref.at[didx.at[0]], add=True)

        plsc.subcore_barrier()
        for k in range(RPS // ZROWS):
            r0 = sub * RPS + k * ZROWS
            pltpu.sync_copy(acc_ref.at[pl.ds(r0, ZROWS)], zbuf)
            pltpu.sync_copy(zbuf, out_hbm.at[core, pl.ds(r0, ZROWS)])


BLK = 1024


def _dinv_of(degp_ref):
    deg = degp_ref[0, :, 0:1] + degp_ref[1, :, 0:1] + 1.0
    return lax.rsqrt(deg)


def _tc1_body(x_ref, w_ref, degp_ref, xs_ref):
    dinv = _dinv_of(degp_ref)
    xw = jnp.dot(x_ref[...], w_ref[...], preferred_element_type=jnp.float32)
    xs_ref[...] = xw * dinv


def _tc2_body(p_ref, xs1_ref, degp_ref, b_ref, w_ref, xs2_ref):
    dinv = _dinv_of(degp_ref)
    s = p_ref[0] + p_ref[1] + xs1_ref[...]
    h = jnp.maximum(dinv * s + b_ref[...], 0.0)
    hw = jnp.dot(h, w_ref[...], preferred_element_type=jnp.float32)
    xs2_ref[...] = hw * dinv


def _tc3_body(p_ref, xs2_ref, degp_ref, b_ref, out_ref):
    dinv = _dinv_of(degp_ref)
    s = p_ref[0] + p_ref[1] + xs2_ref[...]
    out_ref[...] = dinv * s + b_ref[...]


_row_spec = pl.BlockSpec((BLK, D), lambda i: (i, 0))
_p_spec = pl.BlockSpec((NC, BLK, D), lambda i: (0, i, 0))
_degp_spec = pl.BlockSpec((NC, BLK, DEGW), lambda i: (0, i, 0))
_w_spec = pl.BlockSpec((D, D), lambda i: (0, 0))
_b_spec = pl.BlockSpec((1, D), lambda i: (0, 0))

_tc1 = pl.pallas_call(
    _tc1_body,
    grid=(NPAD // BLK,),
    in_specs=[_row_spec, _w_spec, _degp_spec],
    out_specs=_row_spec,
    out_shape=jax.ShapeDtypeStruct((NPAD, D), jnp.float32),
)

_tc2 = pl.pallas_call(
    _tc2_body,
    grid=(NPAD // BLK,),
    in_specs=[_p_spec, _row_spec, _degp_spec, _b_spec, _w_spec],
    out_specs=_row_spec,
    out_shape=jax.ShapeDtypeStruct((NPAD, D), jnp.float32),
)

_tc3 = pl.pallas_call(
    _tc3_body,
    grid=(NPAD // BLK,),
    in_specs=[_p_spec, _row_spec, _degp_spec, _b_spec],
    out_specs=_row_spec,
    out_shape=jax.ShapeDtypeStruct((NPAD, D), jnp.float32),
)


def kernel(x, edge_index, W1, b1, W2, b2):
    src = edge_index[0].astype(jnp.int32)
    dst = edge_index[1].astype(jnp.int32)
    b1r = b1.reshape(1, D)
    b2r = b2.reshape(1, D)
    xp = jnp.pad(x, ((0, NPAD - N), (0, 0)))

    degp = _deg_kernel(dst)
    xs1 = _tc1(xp, W1, degp)
    p1 = _agg_kernel(src, dst, xs1)
    xs2 = _tc2(p1, xs1, degp, b1r, W2)
    p2 = _agg_kernel(src, dst, xs2)
    return _tc3(p2, xs2, degp, b2r)[:N]
